# baseline (speedup 1.0000x reference)
import functools

import jax
import jax.numpy as jnp
from jax import lax
from jax.experimental import pallas as pl
from jax.experimental.pallas import tpu as pltpu
from jax.experimental.pallas import tpu_sc as plsc

_W = 128
_KB = _W // 16


@jax.jit
def _gather_t(table2, idx_t):
    R, TW = table2.shape
    J, I = idx_t.shape
    D = TW // 2
    info = plsc.get_sparse_core_info()
    NC, NS = info.num_cores, info.num_subcores
    NW = NC * NS
    assert I == NW * _W and J >= 4 and J % 2 == 0
    n_groups = J // 2

    mesh = plsc.VectorSubcoreMesh(core_axis_name="c", subcore_axis_name="s")

    @functools.partial(
        pl.kernel,
        mesh=mesh,
        out_type=jax.ShapeDtypeStruct((J, D, I), jnp.float32),
        scratch_types=[
            pltpu.VMEM((J, _W), jnp.int32),
            [pltpu.VMEM((_W,), jnp.int32)] * 2,
            [pltpu.VMEM((_W,), jnp.int32)] * 2,
            [pltpu.VMEM((_W, TW), jnp.float32)] * 2,
            [pltpu.VMEM((D, _W), jnp.float32)] * 2,
            [pltpu.SemaphoreType.DMA] * 2,
            [pltpu.SemaphoreType.DMA] * 2,
        ],
        compiler_params=pltpu.CompilerParams(
            use_tc_tiling_on_sc=True, needs_layout_passes=False),
    )
    def k(tbl, idxs, out, idx_v, i2, par, rows, tv, gs, ws):
        wid = lax.axis_index("s") * NC + lax.axis_index("c")
        i0 = wid * _W
        pltpu.sync_copy(idxs.at[:, pl.ds(i0, _W)], idx_v)
        iota16 = lax.iota(jnp.int32, 16)

        def prep_and_fire(j, p):
            jrow = idx_v.at[j]
            for kb in range(_KB):
                v = jrow[pl.ds(kb * 16, 16)]
                i2[p][pl.ds(kb * 16, 16)] = jnp.right_shift(v, 1)
                par[p][pl.ds(kb * 16, 16)] = jnp.bitwise_and(v, 1) * D
            pltpu.make_async_copy(tbl.at[i2[p]], rows[p], gs[p]).start()

        def gwait(p):
            pltpu.make_async_copy(tbl.at[i2[p]], rows[p], gs[p]).wait()

        def transpose(p):
            for kb in range(_KB):
                rid = iota16 + (kb * 16)
                p64 = par[p][pl.ds(kb * 16, 16)]

                def cbody(cg, carry):
                    for c8 in range(8):
                        c = cg * 8 + c8
                        vals = plsc.load_gather(rows[p], [rid, p64 + c])
                        tv[p][c, pl.ds(kb * 16, 16)] = vals
                    return carry

                lax.fori_loop(0, D // 8, cbody, 0, unroll=False)

        def wdesc(j, p):
            return pltpu.make_async_copy(
                tv[p], out.at[j, :, pl.ds(i0, _W)], ws[p])

        prep_and_fire(0, 0)
        prep_and_fire(1, 1)

        def body(g, carry):
            for p in (0, 1):
                j = 2 * g + p
                gwait(p)

                @pl.when(j >= 2)
                def _():
                    wdesc(j - 2, p).wait()

                transpose(p)
                wdesc(j, p).start()

                @pl.when(j + 2 < J)
                def _():
                    prep_and_fire(j + 2, p)

            return carry

        lax.fori_loop(0, n_groups, body, 0, unroll=False)
        wdesc(J - 2, 0).wait()
        wdesc(J - 1, 1).wait()

    return k(table2, idx_t)


def kernel(input_, indices):
    V, D = input_.shape
    table2 = input_.reshape(V // 2, 2 * D)
    out_t = _gather_t(table2, indices.T)
    return out_t.transpose(2, 0, 1)
